# baseline (speedup 1.0000x reference)
import math

import jax
import jax.numpy as jnp
import numpy as np
from jax.experimental import pallas as pl
from jax.experimental.pallas import tpu as pltpu

H, W = 512, 1024


def _row_kernel_tables(h):
    thetas = np.linspace(0.5, h - 0.5, num=h) * math.pi / h
    weight = 1.0 / np.sin(thetas)
    residual = weight % 2
    mask = residual >= 1
    residual[mask] -= 1
    residual[~mask] += 1
    n_ones = (weight - residual).astype(np.int32)
    edge_values = ((weight - n_ones) / 2.0).astype(np.float32)
    lengths = (n_ones + 2).astype(np.int32)
    return lengths, edge_values


_L_np, _E_np = _row_kernel_tables(H)


def _sample_contrib(yg_ref, yp_ref, lt_ref, et_ref, i):
    f = yg_ref.shape[2]
    hp = jnp.float32

    xs_row = jnp.rint(yg_ref[i, 0:1, :] * (W - 1))
    ys_row = jnp.rint(yg_ref[i, 1:2, :] * (H - 1))

    jj = jax.lax.broadcasted_iota(jnp.int32, (f, f), 0)
    kk = jax.lax.broadcasted_iota(jnp.int32, (f, f), 1)
    ident = (jj == kk).astype(hp)
    xs_col = jnp.sum(ident * xs_row, axis=1, keepdims=True)
    ys_col = jnp.sum(ident * ys_row, axis=1, keepdims=True)
    xs_ci = xs_col.astype(jnp.int32)
    ys_ci = ys_col.astype(jnp.int32)

    iota_h_col = jax.lax.broadcasted_iota(jnp.int32, (f, H), 1)
    e_sel = (ys_ci == iota_h_col).astype(hp)

    len_col = jnp.sum(e_sel * lt_ref[:, :], axis=1, keepdims=True)
    edge_col = jnp.sum(e_sel * et_ref[:, :], axis=1, keepdims=True)
    len_ci = len_col.astype(jnp.int32)

    pole = (ys_ci == 0) | (ys_ci == H - 1)
    len_ci = jnp.where(pole, W, len_ci)
    edge_col = jnp.where(pole, 1.0, edge_col)

    left = jnp.where(pole, 0, xs_ci - len_ci // 2)
    pw = jax.lax.broadcasted_iota(jnp.int32, (f, W), 1)
    d = jax.lax.rem(pw - left + 2 * W, W)
    cov = (d < len_ci).astype(hp)
    vals = jnp.where((d == 0) | (d == len_ci - 1), edge_col, 1.0)

    later_same = ((kk > jj) & (ys_ci == ys_row)).astype(jnp.bfloat16)
    later_cov = jax.lax.dot_general(
        later_same, cov.astype(jnp.bfloat16), (((1,), (0,)), ((), ())),
        preferred_element_type=hp)
    weights = cov * (later_cov < 0.5).astype(hp) * vals

    yp = yp_ref[i]
    yp_hi = yp.astype(jnp.bfloat16)
    yp_lo = (yp - yp_hi.astype(hp)).astype(jnp.bfloat16)
    e16 = e_sel.astype(jnp.bfloat16)
    g = (jax.lax.dot_general(e16, yp_hi, (((1,), (0,)), ((), ())),
                             preferred_element_type=hp)
         + jax.lax.dot_general(e16, yp_lo, (((1,), (0,)), ((), ())),
                               preferred_element_type=hp))

    return jnp.sum(weights * g, dtype=hp)


def _fixation_loss_kernel(yg_ref, yp_ref, lt_ref, et_ref, eps_ref, out_ref):
    b = pl.program_id(0)
    nb = pl.num_programs(0)
    f = yg_ref.shape[2]
    spb = yg_ref.shape[0]
    hp = jnp.float32

    s = jnp.float32(0.0)
    for i in range(spb):
        s = s + _sample_contrib(yg_ref, yp_ref, lt_ref, et_ref, i)

    @pl.when(b == 0)
    def _():
        out_ref[:, :] = jnp.zeros_like(out_ref)

    out_ref[:, :] += jnp.reshape(s, (1, 1))

    @pl.when(b == nb - 1)
    def _():
        fc = jnp.full((1, 1), float(f), dtype=hp)
        eps_v = eps_ref[:, :]
        nf = jnp.where(fc < eps_v, eps_v, fc)
        out_ref[:, :] = out_ref[:, :] / (nf * float(nb * spb))


def kernel(y_pred, y_gt, eps=1e-05):
    b, _, h, w = y_pred.shape
    f = y_gt.shape[1]
    yp = y_pred.reshape(b, h, w)
    yg = jnp.transpose(y_gt, (0, 2, 1))
    lt = jnp.asarray(_L_np, dtype=jnp.float32).reshape(1, h)
    et = jnp.asarray(_E_np, dtype=jnp.float32).reshape(1, h)
    eps_a = jnp.asarray(eps, dtype=jnp.float32).reshape(1, 1)

    spb = 8
    out = pl.pallas_call(
        _fixation_loss_kernel,
        grid=(b // spb,),
        in_specs=[
            pl.BlockSpec((spb, 2, f), lambda i: (i, 0, 0)),
            pl.BlockSpec((spb, h, w), lambda i: (i, 0, 0)),
            pl.BlockSpec((1, h), lambda i: (0, 0)),
            pl.BlockSpec((1, h), lambda i: (0, 0)),
            pl.BlockSpec((1, 1), lambda i: (0, 0)),
        ],
        out_specs=pl.BlockSpec((1, 1), lambda i: (0, 0)),
        out_shape=jax.ShapeDtypeStruct((1, 1), jnp.float32),
    )(yg, yp, lt, et, eps_a)
    return jnp.reshape(out, ())
